# baseline (speedup 1.0000x reference)
import functools

import jax
import jax.numpy as jnp
from jax import lax
from jax.experimental import pallas as pl
from jax.experimental.pallas import tpu as pltpu
from jax.experimental.pallas import tpu_sc as plsc

_B = 262144
_N_FREQ = 12
_N_LEVELS = 8
_BASE_RES = 16
_T = 2 ** 19
_HASH = -1640531535

_NW = 32
_PPW = _B // _NW
_CH = 2048
_NCHUNK = _PPW // _CH
_NV = _CH // 16
_GSZ = _CH * 4

_BLK = 4096


def _sc_body(u_hbm, v_hbm, l_hbm, tab0_hbm, tab1_hbm, s0_hbm, s1_hbm,
             u_v, v_v, l_v, idx0_v, idx1_v, w0_v, w1_v, g0_v, g1_v,
             s0_v, s1_v, sem):
    wid = lax.axis_index("s") * 2 + lax.axis_index("c")
    for chunk in range(_NCHUNK):
        base = wid * _PPW + chunk * _CH
        pltpu.sync_copy(u_hbm.at[pl.ds(base, _CH)], u_v)
        pltpu.sync_copy(v_hbm.at[pl.ds(base, _CH)], v_v)
        pltpu.sync_copy(l_hbm.at[pl.ds(base, _CH)], l_v)

        @pl.loop(0, _NV)
        def _(j):
            sl = pl.ds(j * 16, 16)
            uu = u_v[sl]
            vv = v_v[sl]
            ll = l_v[sl]
            m = jnp.minimum(ll * 7.0, 7.0)
            c0 = ((7.0 - m) * 2.0).astype(jnp.int32)
            for p, idx_v, w_v in ((0, idx0_v, w0_v), (1, idx1_v, w1_v)):
                lvl = lax.shift_right_logical(c0 + (1 - p), 1)
                res = lax.shift_left(jnp.int32(_BASE_RES),
                                     lvl).astype(jnp.float32)
                pux = uu * res
                pvy = vv * res
                ix = pux.astype(jnp.int32)
                iy = pvy.astype(jnp.int32)
                fx = pux - ix.astype(jnp.float32)
                fy = pvy - iy.astype(jnp.float32)
                rowbase = lvl * jnp.int32(_T)
                for dx in range(2):
                    cx = ix + dx if dx else ix
                    wx = fx if dx else 1.0 - fx
                    for dy in range(2):
                        cy = iy + dy if dy else iy
                        wy = fy if dy else 1.0 - fy
                        h = lax.bitwise_xor(cx, cy * jnp.int32(_HASH))
                        hidx = lax.bitwise_and(h, jnp.int32(_T - 1))
                        k = dx * 2 + dy
                        off = (j * 4 + k) * 16
                        idx_v[pl.ds(off, 16)] = rowbase + hidx
                        w_v[pl.ds(off, 16)] = wx * wy

        dma0 = pltpu.async_copy(tab0_hbm.at[idx0_v], g0_v, sem)
        dma1 = pltpu.async_copy(tab1_hbm.at[idx1_v], g1_v, sem)
        dma0.wait()
        dma1.wait()

        @pl.loop(0, _NV)
        def _(j):
            o = j * 64

            def wg(w_v, g_v, k):
                return w_v[pl.ds(o + k * 16, 16)] * g_v[pl.ds(o + k * 16, 16)]

            acc0 = (wg(w0_v, g0_v, 0) + wg(w0_v, g0_v, 1)
                    + wg(w0_v, g0_v, 2) + wg(w0_v, g0_v, 3))
            acc1 = (wg(w1_v, g1_v, 0) + wg(w1_v, g1_v, 1)
                    + wg(w1_v, g1_v, 2) + wg(w1_v, g1_v, 3))
            ll = l_v[pl.ds(j * 16, 16)]
            m = jnp.minimum(ll * 7.0, 7.0)
            c0 = ((7.0 - m) * 2.0).astype(jnp.int32)
            even = lax.bitwise_and(c0, 1) == 0
            s0_v[pl.ds(j * 16, 16)] = jnp.where(even, acc0, acc1)
            s1_v[pl.ds(j * 16, 16)] = jnp.where(even, acc1, acc0)

        pltpu.sync_copy(s0_v, s0_hbm.at[pl.ds(base, _CH)])
        pltpu.sync_copy(s1_v, s1_hbm.at[pl.ds(base, _CH)])


@functools.cache
def _sc_sample_fn():
    return functools.partial(
        pl.kernel,
        mesh=plsc.VectorSubcoreMesh(core_axis_name="c", subcore_axis_name="s"),
        out_type=[jax.ShapeDtypeStruct((_B,), jnp.float32),
                  jax.ShapeDtypeStruct((_B,), jnp.float32)],
        scratch_types=[
            pltpu.VMEM((_CH,), jnp.float32),
            pltpu.VMEM((_CH,), jnp.float32),
            pltpu.VMEM((_CH,), jnp.float32),
            pltpu.VMEM((_GSZ,), jnp.int32),
            pltpu.VMEM((_GSZ,), jnp.int32),
            pltpu.VMEM((_GSZ,), jnp.float32),
            pltpu.VMEM((_GSZ,), jnp.float32),
            pltpu.VMEM((_GSZ,), jnp.float32),
            pltpu.VMEM((_GSZ,), jnp.float32),
            pltpu.VMEM((_CH,), jnp.float32),
            pltpu.VMEM((_CH,), jnp.float32),
            pltpu.SemaphoreType.DMA,
        ],
    )(_sc_body)


_DT_C = 65536


def _detile_body(t_ref, o_ref):
    o_ref[...] = t_ref[...].reshape(_N_LEVELS, _DT_C // 128, 128)


def _detile_plane(tp):
    out = pl.pallas_call(
        _detile_body,
        grid=(_T // _DT_C,),
        in_specs=[pl.BlockSpec((_N_LEVELS, _DT_C), lambda c: (0, c))],
        out_specs=pl.BlockSpec((_N_LEVELS, _DT_C // 128, 128),
                               lambda c: (0, c, 0)),
        out_shape=jax.ShapeDtypeStruct(
            (_N_LEVELS, _T // 128, 128), jnp.float32),
    )(tp)
    return out.reshape(-1)


def _tc_body(x_ref, s0_ref, s1_ref, w0uv_ref, w0t_ref,
             w1_ref, w2_ref, o_ref):
    xb = x_ref[...]
    u = xb[:, 0:1]
    v = xb[:, 1:2]
    lod = xb[:, 2:3]
    fi = lax.broadcasted_iota(jnp.int32, (1, _N_FREQ), 1)
    freqs = lax.shift_left(jnp.int32(1), fi).astype(jnp.float32)
    xu = u * freqs
    pe_u = jnp.abs(xu - jnp.floor(xu + 0.5)) * 2.0
    xv = v * freqs
    pe_v = jnp.abs(xv - jnp.floor(xv + 0.5)) * 2.0
    pe = jnp.concatenate([pe_u, pe_v], axis=1)
    w0t = w0t_ref[...]
    h = (jnp.dot(pe, w0uv_ref[...], preferred_element_type=jnp.float32)
         + jnp.dot(s0_ref[...], w0t[0:1, :],
                   preferred_element_type=jnp.float32)
         + jnp.dot(s1_ref[...], w0t[1:2, :],
                   preferred_element_type=jnp.float32)
         + jnp.dot(lod, w0t[2:3, :], preferred_element_type=jnp.float32))
    h = jnp.where(h > 0, h, 0.01 * h)
    h = jnp.dot(h, w1_ref[...], preferred_element_type=jnp.float32)
    h = jnp.where(h > 0, h, 0.01 * h)
    o = jnp.dot(h, w2_ref[...], preferred_element_type=jnp.float32)
    o_ref[...] = jnp.where(o > 0, o, 0.01 * o)


def _tc_mlp(x, s0, s1, w0uv, w0t, W1, W2):
    rep = lambda i: (0, 0)
    return pl.pallas_call(
        _tc_body,
        grid=(_B // _BLK,),
        in_specs=[
            pl.BlockSpec((_BLK, 3), lambda i: (i, 0)),
            pl.BlockSpec((_BLK, 1), lambda i: (i, 0)),
            pl.BlockSpec((_BLK, 1), lambda i: (i, 0)),
            pl.BlockSpec((2 * _N_FREQ, 64), rep),
            pl.BlockSpec((3, 64), rep),
            pl.BlockSpec((64, 64), rep),
            pl.BlockSpec((64, 3), rep),
        ],
        out_specs=pl.BlockSpec((_BLK, 3), lambda i: (i, 0)),
        out_shape=jax.ShapeDtypeStruct((_B, 3), jnp.float32),
    )(x, s0, s1, w0uv, w0t, W1, W2)


@jax.jit
def kernel(x, table, W0, W1, W2):
    u = x[:, 0]
    v = x[:, 1]
    lod = x[:, 2]
    tab0 = _detile_plane(table[:, :, 0])
    tab1 = _detile_plane(table[:, :, 1])
    s0, s1 = _sc_sample_fn()(u, v, lod, tab0, tab1)
    perm = jnp.asarray(list(range(0, 2 * _N_FREQ, 2))
                       + list(range(1, 2 * _N_FREQ, 2)))
    w0uv = W0[perm]
    w0t = W0[2 * _N_FREQ:]
    return _tc_mlp(x, s0.reshape(_B, 1), s1.reshape(_B, 1),
                   w0uv, w0t, W1, W2)


# speedup vs baseline: 1.0686x; 1.0686x over previous
import functools

import jax
import jax.numpy as jnp
from jax import lax
from jax.experimental import pallas as pl
from jax.experimental.pallas import tpu as pltpu
from jax.experimental.pallas import tpu_sc as plsc

_B = 262144
_N_FREQ = 12
_N_LEVELS = 8
_BASE_RES = 16
_T = 2 ** 19
_HASH = -1640531535

_NW = 32
_PPW = _B // _NW
_CH = 1024
_NCHUNK = _PPW // _CH
_NV = _CH // 16
_GSZ = _CH * 4

_BLK = 8192


def _sc_body(u_hbm, v_hbm, l_hbm, tab0_hbm, tab1_hbm, s0_hbm, s1_hbm,
             ua_v, va_v, la_v, i0a_v, i1a_v, w0a_v, w1a_v, g0a_v, g1a_v,
             ub_v, vb_v, lb_v, i0b_v, i1b_v, w0b_v, w1b_v, g0b_v, g1b_v,
             s0_v, s1_v, sema, semb):
    bufs = (
        (ua_v, va_v, la_v, i0a_v, i1a_v, w0a_v, w1a_v, g0a_v, g1a_v, sema),
        (ub_v, vb_v, lb_v, i0b_v, i1b_v, w0b_v, w1b_v, g0b_v, g1b_v, semb),
    )
    wid = lax.axis_index("s") * 2 + lax.axis_index("c")

    def load(chunk, bs):
        u_v, v_v, l_v = bs[0], bs[1], bs[2]
        base = wid * _PPW + chunk * _CH
        pltpu.sync_copy(u_hbm.at[pl.ds(base, _CH)], u_v)
        pltpu.sync_copy(v_hbm.at[pl.ds(base, _CH)], v_v)
        pltpu.sync_copy(l_hbm.at[pl.ds(base, _CH)], l_v)

    def compute(bs):
        u_v, v_v, l_v, idx0_v, idx1_v, w0_v, w1_v = bs[0:7]

        @pl.loop(0, _NV)
        def _(j):
            sl = pl.ds(j * 16, 16)
            uu = u_v[sl]
            vv = v_v[sl]
            ll = l_v[sl]
            m = jnp.minimum(ll * 7.0, 7.0)
            c0 = ((7.0 - m) * 2.0).astype(jnp.int32)
            for p, idx_v, w_v in ((0, idx0_v, w0_v), (1, idx1_v, w1_v)):
                lvl = lax.shift_right_logical(c0 + (1 - p), 1)
                res = lax.shift_left(jnp.int32(_BASE_RES),
                                     lvl).astype(jnp.float32)
                pux = uu * res
                pvy = vv * res
                ix = pux.astype(jnp.int32)
                iy = pvy.astype(jnp.int32)
                fx = pux - ix.astype(jnp.float32)
                fy = pvy - iy.astype(jnp.float32)
                rowbase = lvl * jnp.int32(_T)
                for dx in range(2):
                    cx = ix + dx if dx else ix
                    wx = fx if dx else 1.0 - fx
                    for dy in range(2):
                        cy = iy + dy if dy else iy
                        wy = fy if dy else 1.0 - fy
                        h = lax.bitwise_xor(cx, cy * jnp.int32(_HASH))
                        hidx = lax.bitwise_and(h, jnp.int32(_T - 1))
                        k = dx * 2 + dy
                        off = (j * 4 + k) * 16
                        idx_v[pl.ds(off, 16)] = rowbase + hidx
                        w_v[pl.ds(off, 16)] = wx * wy

    def gather_start(bs):
        return (pltpu.async_copy(tab0_hbm.at[bs[3]], bs[7], bs[9]),
                pltpu.async_copy(tab1_hbm.at[bs[4]], bs[8], bs[9]))

    def accumulate(chunk, bs):
        l_v, w0_v, w1_v, g0_v, g1_v = bs[2], bs[5], bs[6], bs[7], bs[8]
        base = wid * _PPW + chunk * _CH

        @pl.loop(0, _NV)
        def _(j):
            o = j * 64

            def wg(w_v, g_v, k):
                return w_v[pl.ds(o + k * 16, 16)] * g_v[pl.ds(o + k * 16, 16)]

            acc0 = (wg(w0_v, g0_v, 0) + wg(w0_v, g0_v, 1)
                    + wg(w0_v, g0_v, 2) + wg(w0_v, g0_v, 3))
            acc1 = (wg(w1_v, g1_v, 0) + wg(w1_v, g1_v, 1)
                    + wg(w1_v, g1_v, 2) + wg(w1_v, g1_v, 3))
            ll = l_v[pl.ds(j * 16, 16)]
            m = jnp.minimum(ll * 7.0, 7.0)
            c0 = ((7.0 - m) * 2.0).astype(jnp.int32)
            even = lax.bitwise_and(c0, 1) == 0
            s0_v[pl.ds(j * 16, 16)] = jnp.where(even, acc0, acc1)
            s1_v[pl.ds(j * 16, 16)] = jnp.where(even, acc1, acc0)

        pltpu.sync_copy(s0_v, s0_hbm.at[pl.ds(base, _CH)])
        pltpu.sync_copy(s1_v, s1_hbm.at[pl.ds(base, _CH)])

    load(0, bufs[0])
    compute(bufs[0])
    dmas = {0: gather_start(bufs[0])}
    for chunk in range(_NCHUNK):
        cur = bufs[chunk % 2]
        if chunk + 1 < _NCHUNK:
            nxt = bufs[(chunk + 1) % 2]
            load(chunk + 1, nxt)
            compute(nxt)
            dmas[chunk + 1] = gather_start(nxt)
        d0, d1 = dmas.pop(chunk)
        d0.wait()
        d1.wait()
        accumulate(chunk, cur)


@functools.cache
def _sc_sample_fn():
    return functools.partial(
        pl.kernel,
        mesh=plsc.VectorSubcoreMesh(core_axis_name="c", subcore_axis_name="s"),
        out_type=[jax.ShapeDtypeStruct((_B,), jnp.float32),
                  jax.ShapeDtypeStruct((_B,), jnp.float32)],
        scratch_types=(
            [pltpu.VMEM((_CH,), jnp.float32),
             pltpu.VMEM((_CH,), jnp.float32),
             pltpu.VMEM((_CH,), jnp.float32),
             pltpu.VMEM((_GSZ,), jnp.int32),
             pltpu.VMEM((_GSZ,), jnp.int32),
             pltpu.VMEM((_GSZ,), jnp.float32),
             pltpu.VMEM((_GSZ,), jnp.float32),
             pltpu.VMEM((_GSZ,), jnp.float32),
             pltpu.VMEM((_GSZ,), jnp.float32)] * 2
            + [pltpu.VMEM((_CH,), jnp.float32),
               pltpu.VMEM((_CH,), jnp.float32),
               pltpu.SemaphoreType.DMA,
               pltpu.SemaphoreType.DMA]),
    )(_sc_body)


_DT_C = 65536


def _detile_body(t_ref, o_ref):
    o_ref[...] = t_ref[...].reshape(_N_LEVELS, _DT_C // 128, 128)


def _detile_plane(tp):
    out = pl.pallas_call(
        _detile_body,
        grid=(_T // _DT_C,),
        in_specs=[pl.BlockSpec((_N_LEVELS, _DT_C), lambda c: (0, c))],
        out_specs=pl.BlockSpec((_N_LEVELS, _DT_C // 128, 128),
                               lambda c: (0, c, 0)),
        out_shape=jax.ShapeDtypeStruct(
            (_N_LEVELS, _T // 128, 128), jnp.float32),
    )(tp)
    return out.reshape(-1)


def _tc_body(x_ref, s0_ref, s1_ref, w0uv_ref, w0t_ref,
             w1_ref, w2_ref, o_ref):
    xb = x_ref[...]
    u = xb[:, 0:1]
    v = xb[:, 1:2]
    lod = xb[:, 2:3]
    fi = lax.broadcasted_iota(jnp.int32, (1, _N_FREQ), 1)
    freqs = lax.shift_left(jnp.int32(1), fi).astype(jnp.float32)
    xu = u * freqs
    pe_u = jnp.abs(xu - jnp.floor(xu + 0.5)) * 2.0
    xv = v * freqs
    pe_v = jnp.abs(xv - jnp.floor(xv + 0.5)) * 2.0
    pe = jnp.concatenate([pe_u, pe_v], axis=1)
    w0t = w0t_ref[...]
    h = (jnp.dot(pe, w0uv_ref[...], preferred_element_type=jnp.float32)
         + jnp.dot(s0_ref[...], w0t[0:1, :],
                   preferred_element_type=jnp.float32)
         + jnp.dot(s1_ref[...], w0t[1:2, :],
                   preferred_element_type=jnp.float32)
         + jnp.dot(lod, w0t[2:3, :], preferred_element_type=jnp.float32))
    h = jnp.where(h > 0, h, 0.01 * h)
    h = jnp.dot(h, w1_ref[...], preferred_element_type=jnp.float32)
    h = jnp.where(h > 0, h, 0.01 * h)
    o = jnp.dot(h, w2_ref[...], preferred_element_type=jnp.float32)
    o_ref[...] = jnp.where(o > 0, o, 0.01 * o)


def _tc_mlp(x, s0, s1, w0uv, w0t, W1, W2):
    rep = lambda i: (0, 0)
    return pl.pallas_call(
        _tc_body,
        grid=(_B // _BLK,),
        in_specs=[
            pl.BlockSpec((_BLK, 3), lambda i: (i, 0)),
            pl.BlockSpec((_BLK, 1), lambda i: (i, 0)),
            pl.BlockSpec((_BLK, 1), lambda i: (i, 0)),
            pl.BlockSpec((2 * _N_FREQ, 64), rep),
            pl.BlockSpec((3, 64), rep),
            pl.BlockSpec((64, 64), rep),
            pl.BlockSpec((64, 3), rep),
        ],
        out_specs=pl.BlockSpec((_BLK, 3), lambda i: (i, 0)),
        out_shape=jax.ShapeDtypeStruct((_B, 3), jnp.float32),
    )(x, s0, s1, w0uv, w0t, W1, W2)


@jax.jit
def kernel(x, table, W0, W1, W2):
    u = x[:, 0]
    v = x[:, 1]
    lod = x[:, 2]
    tab0 = _detile_plane(table[:, :, 0])
    tab1 = _detile_plane(table[:, :, 1])
    s0, s1 = _sc_sample_fn()(u, v, lod, tab0, tab1)
    perm = jnp.asarray(list(range(0, 2 * _N_FREQ, 2))
                       + list(range(1, 2 * _N_FREQ, 2)))
    w0uv = W0[perm]
    w0t = W0[2 * _N_FREQ:]
    return _tc_mlp(x, s0.reshape(_B, 1), s1.reshape(_B, 1),
                   w0uv, w0t, W1, W2)
